# baseline (speedup 1.0000x reference)
import functools

import jax
import jax.numpy as jnp
from jax import lax
from jax.experimental import pallas as pl
from jax.experimental.pallas import tpu as pltpu
from jax.experimental.pallas import tpu_sc as plsc

N = 10000
E = 320000
D = 128
C = 16

NC = 2
NS = 16
NW = NC * NS
CH = 128
NBLK = E // CH
NG = CH // 16


def _tables_kernel(x_ref, w_ref, b_ref, y1_ref, y2_ref):
    x = x_ref[...]
    w = w_ref[...]
    dn = (((1,), (1,)), ((), ()))
    y1_ref[...] = lax.dot_general(x, w[:, :D], dn,
                                  preferred_element_type=jnp.float32)
    y2_ref[...] = lax.dot_general(x, w[:, D:], dn,
                                  preferred_element_type=jnp.float32) + b_ref[...]


def _edge_kernel(y1_hbm, y2_hbm, ei_hbm, out_hbm,
                 idx, r1, r2, tbuf, sem_i, sem_g, sem_s):
    wid = lax.axis_index("s") * NC + lax.axis_index("c")
    kmax = lax.div(NBLK - wid + NW - 1, NW)

    def bidx(k):
        return wid + k * NW

    def fire_idx(k, b):
        off = bidx(k) * CH
        pltpu.async_copy(ei_hbm.at[pl.ds(0, 2), pl.ds(off, CH)],
                         idx.at[b], sem_i.at[b])

    def wait_idx(k, b):
        off = bidx(k) * CH
        pltpu.make_async_copy(ei_hbm.at[pl.ds(0, 2), pl.ds(off, CH)],
                              idx.at[b], sem_i.at[b]).wait()

    def fire_g(b):
        pltpu.async_copy(y1_hbm.at[idx.at[b, 0]], r1.at[b], sem_g.at[b])
        pltpu.async_copy(y2_hbm.at[idx.at[b, 1]], r2.at[b], sem_g.at[b])

    def wait_g(b):
        pltpu.make_async_copy(y1_hbm.at[idx.at[b, 0]], r1.at[b],
                              sem_g.at[b]).wait()
        pltpu.make_async_copy(y2_hbm.at[idx.at[b, 1]], r2.at[b],
                              sem_g.at[b]).wait()

    def store(k, b):
        ci = bidx(k)
        pltpu.async_copy(tbuf.at[b, 0], out_hbm.at[0, ci], sem_s.at[b])
        pltpu.async_copy(tbuf.at[b, 1], out_hbm.at[1, ci], sem_s.at[b])

    def wait_store(k, b):
        ci = bidx(k)
        pltpu.make_async_copy(tbuf.at[b, 0], out_hbm.at[0, ci],
                              sem_s.at[b]).wait()
        pltpu.make_async_copy(tbuf.at[b, 1], out_hbm.at[1, ci],
                              sem_s.at[b]).wait()

    fire_idx(0, 0)
    fire_idx(1, 1)
    fire_idx(2, 2)
    wait_idx(0, 0)
    fire_g(0)
    wait_idx(1, 1)
    fire_g(1)

    lane = jnp.arange(16, dtype=jnp.int32)
    e_idx = [lane + 16 * g for g in range(NG)]
    c_vec = [jnp.full((16,), c, dtype=jnp.int32) for c in range(C)]

    def body(k, _):
        b3 = lax.rem(k, 3)
        b2 = lax.rem(k, 2)

        wait_g(b3)

        @pl.when(k + 2 < kmax)
        def _():
            wait_idx(k + 2, lax.rem(k + 2, 3))
            fire_g(lax.rem(k + 2, 3))

        @pl.when(k + 3 < kmax)
        def _():
            fire_idx(k + 3, b3)

        @pl.when(k >= 2)
        def _():
            wait_store(k - 2, b2)

        b_vec = jnp.full((16,), b3, dtype=jnp.int32)
        for c in range(C):
            for g in range(NG):
                v = (plsc.load_gather(r1, [b_vec, e_idx[g], c_vec[c]])
                     + plsc.load_gather(r2, [b_vec, e_idx[g], c_vec[c]]))
                tbuf[b2, c // 8, c % 8, pl.ds(16 * g, 16)] = v

        store(k, b2)
        return 0

    lax.fori_loop(0, kmax, body, 0)

    wait_store(kmax - 2, lax.rem(kmax - 2, 2))
    wait_store(kmax - 1, lax.rem(kmax - 1, 2))


def kernel(x, edge_index, W, b):
    y1, y2 = pl.pallas_call(
        _tables_kernel,
        out_shape=(
            jax.ShapeDtypeStruct((N, C), jnp.float32),
            jax.ShapeDtypeStruct((N, C), jnp.float32),
        ),
    )(x, W, b.reshape(1, C))

    ei = edge_index.astype(jnp.int32)

    mesh = plsc.VectorSubcoreMesh(core_axis_name="c", subcore_axis_name="s",
                                  num_cores=NC, num_subcores=NS)
    out4 = pl.kernel(
        _edge_kernel,
        out_type=jax.ShapeDtypeStruct((2, E // CH, 8, CH), jnp.float32),
        mesh=mesh,
        scratch_types=[
            pltpu.VMEM((3, 2, CH), jnp.int32),
            pltpu.VMEM((3, CH, C), jnp.float32),
            pltpu.VMEM((3, CH, C), jnp.float32),
            pltpu.VMEM((2, 2, 8, CH), jnp.float32),
            pltpu.SemaphoreType.DMA((3,)),
            pltpu.SemaphoreType.DMA((3,)),
            pltpu.SemaphoreType.DMA((2,)),
        ],
        compiler_params=pltpu.CompilerParams(use_tc_tiling_on_sc=False,
                                             needs_layout_passes=False),
    )(y1, y2, ei)

    return out4.transpose(0, 2, 1, 3).reshape(C, E).T


# speedup vs baseline: 1.4747x; 1.4747x over previous
import functools

import jax
import jax.numpy as jnp
from jax import lax
from jax.experimental import pallas as pl
from jax.experimental.pallas import tpu as pltpu
from jax.experimental.pallas import tpu_sc as plsc

N = 10000
E = 320000
D = 128
C = 16

NC = 2
NS = 16
NW = NC * NS
TE = 128
NT = 2
CH = TE * NT
NBLK = E // CH
NG = CH // 16


def _tables_kernel(x_ref, w_ref, b_ref, y1_ref, y2_ref):
    x = x_ref[...]
    w = w_ref[...]
    dn = (((1,), (1,)), ((), ()))
    y1_ref[...] = lax.dot_general(x, w[:, :D], dn,
                                  preferred_element_type=jnp.float32)
    y2_ref[...] = lax.dot_general(x, w[:, D:], dn,
                                  preferred_element_type=jnp.float32) + b_ref[...]


def _edge_kernel(y1_hbm, y2_hbm, ei_hbm, out_hbm,
                 idx, r, tbuf, sem_i, sem_g1, sem_g2, sem_s):
    wid = lax.axis_index("s") * NC + lax.axis_index("c")
    kmax = lax.div(NBLK - wid + NW - 1, NW)

    def bidx(k):
        return wid + k * NW

    def fire_idx(k, b):
        off = bidx(k) * CH
        pltpu.async_copy(ei_hbm.at[pl.ds(0, 2), pl.ds(off, CH)],
                         idx.at[b], sem_i.at[b])

    def wait_idx(k, b):
        off = bidx(k) * CH
        pltpu.make_async_copy(ei_hbm.at[pl.ds(0, 2), pl.ds(off, CH)],
                              idx.at[b], sem_i.at[b]).wait()

    def fire_g1(b):
        for t in range(NT):
            pltpu.async_copy(y1_hbm.at[idx.at[b, 0, pl.ds(t * TE, TE)]],
                             r.at[b, pl.ds(t * TE, TE)], sem_g1.at[b])

    def wait_g1(b):
        for t in range(NT):
            pltpu.make_async_copy(y1_hbm.at[idx.at[b, 0, pl.ds(t * TE, TE)]],
                                  r.at[b, pl.ds(t * TE, TE)],
                                  sem_g1.at[b]).wait()

    def fire_g2(b):
        for t in range(NT):
            pltpu.async_copy(y2_hbm.at[idx.at[b, 1, pl.ds(t * TE, TE)]],
                             r.at[b, pl.ds(t * TE, TE)], sem_g2.at[b],
                             add=True)

    def wait_g2(b):
        for t in range(NT):
            pltpu.make_async_copy(y2_hbm.at[idx.at[b, 1, pl.ds(t * TE, TE)]],
                                  r.at[b, pl.ds(t * TE, TE)],
                                  sem_g2.at[b]).wait()

    def store(k, b):
        ci = bidx(k) * NT
        pltpu.async_copy(tbuf.at[b, 0], out_hbm.at[0, pl.ds(ci, NT)],
                         sem_s.at[b])
        pltpu.async_copy(tbuf.at[b, 1], out_hbm.at[1, pl.ds(ci, NT)],
                         sem_s.at[b])

    def wait_store(k, b):
        ci = bidx(k) * NT
        pltpu.make_async_copy(tbuf.at[b, 0], out_hbm.at[0, pl.ds(ci, NT)],
                              sem_s.at[b]).wait()
        pltpu.make_async_copy(tbuf.at[b, 1], out_hbm.at[1, pl.ds(ci, NT)],
                              sem_s.at[b]).wait()

    fire_idx(0, 0)
    fire_idx(1, 1)
    fire_idx(2, 2)
    wait_idx(0, 0)
    fire_g1(0)
    wait_idx(1, 1)
    fire_g1(1)
    wait_g1(0)
    fire_g2(0)

    lane = jnp.arange(16, dtype=jnp.int32)
    e_idx = [lane + 16 * g for g in range(NG)]
    c_vec = [jnp.full((16,), c, dtype=jnp.int32) for c in range(C)]

    def body(k, _):
        b3 = lax.rem(k, 3)
        b2 = lax.rem(k, 2)

        wait_g2(b3)

        @pl.when(k + 1 < kmax)
        def _():
            wait_g1(lax.rem(k + 1, 3))
            fire_g2(lax.rem(k + 1, 3))

        @pl.when(k + 2 < kmax)
        def _():
            wait_idx(k + 2, lax.rem(k + 2, 3))
            fire_g1(lax.rem(k + 2, 3))

        @pl.when(k + 3 < kmax)
        def _():
            fire_idx(k + 3, b3)

        @pl.when(k >= 2)
        def _():
            wait_store(k - 2, b2)

        b_vec = jnp.full((16,), b3, dtype=jnp.int32)
        for c in range(C):
            for g in range(NG):
                v = plsc.load_gather(r, [b_vec, e_idx[g], c_vec[c]])
                tbuf[b2, c // 8, g // 8, c % 8, pl.ds(16 * (g % 8), 16)] = v

        store(k, b2)
        return 0

    lax.fori_loop(0, kmax, body, 0)

    wait_store(kmax - 2, lax.rem(kmax - 2, 2))
    wait_store(kmax - 1, lax.rem(kmax - 1, 2))


def kernel(x, edge_index, W, b):
    y1, y2 = pl.pallas_call(
        _tables_kernel,
        out_shape=(
            jax.ShapeDtypeStruct((N, C), jnp.float32),
            jax.ShapeDtypeStruct((N, C), jnp.float32),
        ),
    )(x, W, b.reshape(1, C))

    ei = edge_index.astype(jnp.int32)

    mesh = plsc.VectorSubcoreMesh(core_axis_name="c", subcore_axis_name="s",
                                  num_cores=NC, num_subcores=NS)
    out4 = pl.kernel(
        _edge_kernel,
        out_type=jax.ShapeDtypeStruct((2, E // TE, 8, TE), jnp.float32),
        mesh=mesh,
        scratch_types=[
            pltpu.VMEM((3, 2, CH), jnp.int32),
            pltpu.VMEM((3, CH, C), jnp.float32),
            pltpu.VMEM((2, 2, NT, 8, TE), jnp.float32),
            pltpu.SemaphoreType.DMA((3,)),
            pltpu.SemaphoreType.DMA((3,)),
            pltpu.SemaphoreType.DMA((3,)),
            pltpu.SemaphoreType.DMA((2,)),
        ],
        compiler_params=pltpu.CompilerParams(use_tc_tiling_on_sc=False,
                                             needs_layout_passes=False),
    )(y1, y2, ei)

    return out4.transpose(0, 2, 1, 3).reshape(C, E).T
